# Initial kernel scaffold; baseline (speedup 1.0000x reference)
import jax
import jax.numpy as jnp
from jax.experimental import pallas as pl


def kernel(x, l, params):
    raise NotImplementedError("write your pallas kernel here")


# speedup vs baseline: 13.1387x; 13.1387x over previous
import jax
import jax.numpy as jnp
from jax.experimental import pallas as pl

_IMAX = 0x7FFFFFFF
_INTERP = False


def _pack(d2):
    bits = jax.lax.bitcast_convert_type(d2, jnp.int32)
    key = jnp.where(bits < 0, bits ^ jnp.int32(0x7FFFFFFF), bits)
    idx = jax.lax.broadcasted_iota(jnp.int32, d2.shape, d2.ndim - 1)
    return (key & jnp.int32(~0x7FF)) | idx


def _select_topk(P, k):
    sel = jnp.zeros(P.shape, jnp.bool_)
    for _ in range(k):
        m = jnp.min(P, axis=-1, keepdims=True)
        oh = P == m
        sel = jnp.logical_or(sel, oh)
        P = jnp.where(oh, _IMAX, P)
    return sel


def _d2(q, kxt):
    qsq = jnp.sum(q * q, axis=-1, keepdims=True)
    ksq = jnp.sum(kxt * kxt, axis=0, keepdims=True)
    qk = jax.lax.dot_general(q, kxt, (((1,), (0,)), ((), ())),
                             preferred_element_type=jnp.float32)
    return (qsq - 2.0 * qk) + ksq


def _mm(a, b):
    return jnp.dot(a, b, preferred_element_type=jnp.float32)


def _diffconv_stage(feat, xyz, npoint, p, radius):
    B, Nn, C = feat.shape
    stride = Nn // npoint
    new_xyz = xyz[:, ::stride]
    cf = feat[:, ::stride]
    kxt = jnp.swapaxes(xyz, 1, 2)
    Co = p['Wd'].shape[1]
    Mb = min(npoint, 256)
    grid = (B, npoint // Mb)
    wa_t = jnp.swapaxes(p['wa'], 0, 1)
    bias = p['b'].reshape(1, Co)

    def body(q_ref, kxt_ref, feat_ref, cf_ref, wat_ref, wd_ref, ws_ref,
             b_ref, o_ref):
        q = q_ref[0]
        kxt_ = kxt_ref[0]
        ft = feat_ref[0]
        cfb = cf_ref[0]
        d2 = _d2(q, kxt_)
        sel = _select_topk(_pack(d2), 20)
        srow = jax.lax.dot_general(wat_ref[...], ft, (((1,), (1,)), ((), ())),
                                   preferred_element_type=jnp.float32)
        ml = jnp.where(sel, srow - d2 / radius, -jnp.inf)
        rmax = jnp.max(ml, axis=-1, keepdims=True)
        e = jnp.exp(ml - rmax)
        a = e / jnp.sum(e, axis=-1, keepdims=True)
        agg = _mm(a, ft) - cfb
        out = _mm(agg, wd_ref[...]) + _mm(cfb, ws_ref[...]) + b_ref[...]
        o_ref[0] = jax.nn.gelu(out)

    out = pl.pallas_call(
        body,
        grid=grid,
        in_specs=[
            pl.BlockSpec((1, Mb, 3), lambda b, i: (b, i, 0)),
            pl.BlockSpec((1, 3, Nn), lambda b, i: (b, 0, 0)),
            pl.BlockSpec((1, Nn, C), lambda b, i: (b, 0, 0)),
            pl.BlockSpec((1, Mb, C), lambda b, i: (b, i, 0)),
            pl.BlockSpec((1, C), lambda b, i: (0, 0)),
            pl.BlockSpec((C, Co), lambda b, i: (0, 0)),
            pl.BlockSpec((C, Co), lambda b, i: (0, 0)),
            pl.BlockSpec((1, Co), lambda b, i: (0, 0)),
        ],
        out_specs=pl.BlockSpec((1, Mb, Co), lambda b, i: (b, i, 0)),
        out_shape=jax.ShapeDtypeStruct((B, npoint, Co), jnp.float32),
        interpret=_INTERP,
    )(new_xyz, kxt, feat, cf, wa_t, p['Wd'], p['Ws'], bias)
    return out, new_xyz


def _fp_stage(xyz1, xyz2, feat1, feat2, p):
    B, M, C1 = feat1.shape
    _, N2, C2 = feat2.shape
    kxt = jnp.swapaxes(xyz2, 1, 2)
    Co = p['W'].shape[1]
    w_hi = p['W'][:C2]
    w_lo = p['W'][C2:]
    bias = p['b'].reshape(1, Co)
    Mb = min(M, 256)
    grid = (B, M // Mb)

    def body(q_ref, kxt_ref, f2_ref, f1_ref, whi_ref, wlo_ref, b_ref, o_ref):
        d2 = _d2(q_ref[0], kxt_ref[0])
        sel = _select_topk(_pack(d2), 3)
        w = jnp.where(sel, 1.0 / (d2 + 1e-8), 0.0)
        wn = w / jnp.sum(w, axis=-1, keepdims=True)
        interp = _mm(wn, f2_ref[0])
        out = _mm(interp, whi_ref[...]) + _mm(f1_ref[0], wlo_ref[...]) + b_ref[...]
        o_ref[0] = jax.nn.gelu(out)

    return pl.pallas_call(
        body,
        grid=grid,
        in_specs=[
            pl.BlockSpec((1, Mb, 3), lambda b, i: (b, i, 0)),
            pl.BlockSpec((1, 3, N2), lambda b, i: (b, 0, 0)),
            pl.BlockSpec((1, N2, C2), lambda b, i: (b, 0, 0)),
            pl.BlockSpec((1, Mb, C1), lambda b, i: (b, i, 0)),
            pl.BlockSpec((C2, Co), lambda b, i: (0, 0)),
            pl.BlockSpec((C1, Co), lambda b, i: (0, 0)),
            pl.BlockSpec((1, Co), lambda b, i: (0, 0)),
        ],
        out_specs=pl.BlockSpec((1, Mb, Co), lambda b, i: (b, i, 0)),
        out_shape=jax.ShapeDtypeStruct((B, M, Co), jnp.float32),
        interpret=_INTERP,
    )(xyz1, kxt, feat2, feat1, w_hi, w_lo, bias)


def _sa_stage(x, params):
    B, Nn, _ = x.shape
    kxt = jnp.swapaxes(x, 1, 2)
    le0 = params['le0']
    l1 = params['le1']
    w0 = le0['W']
    b0 = le0['b'].reshape(1, -1)
    w1 = l1[0]['W']
    w1r, w1f = w1[:3], w1[3:]
    b1 = l1[0]['b'].reshape(1, -1)
    w2 = l1[1]['W']
    b2 = l1[1]['b'].reshape(1, -1)
    w3 = l1[2]['W']
    b3 = l1[2]['b'].reshape(1, -1)
    C = w0.shape[1]
    Mb = 256
    grid = (B, Nn // Mb)
    R2 = 0.05 * 0.05

    def body(q_ref, kxt_ref, w0_ref, b0_ref, w1r_ref, w1f_ref, b1_ref,
             w2_ref, b2_ref, w3_ref, b3_ref, o_ref):
        q = q_ref[0]
        kxt_ = kxt_ref[0]
        xw1r = jax.lax.dot_general(kxt_, w1r_ref[...], (((0,), (0,)), ((), ())),
                                   preferred_element_type=jnp.float32)
        feat = jax.nn.gelu(
            jax.lax.dot_general(kxt_, w0_ref[...], (((0,), (0,)), ((), ())),
                                preferred_element_type=jnp.float32) + b0_ref[...])
        U = _mm(feat, w1f_ref[...]) + xw1r + b1_ref[...]
        V = _mm(q, w1r_ref[...])
        d2 = _d2(q, kxt_)
        P = _pack(d2)
        g = jnp.full((Mb, C), -jnp.inf, jnp.float32)
        for _ in range(20):
            m = jnp.min(P, axis=-1, keepdims=True)
            oh = P == m
            P = jnp.where(oh, _IMAX, P)
            ohf = jnp.where(oh, 1.0, 0.0)
            Ug = _mm(ohf, U)
            d2i = jnp.sum(jnp.where(oh, d2, 0.0), axis=-1, keepdims=True)
            h = jnp.maximum(Ug - V, 0.0)
            h = jnp.maximum(_mm(h, w2_ref[...]) + b2_ref[...], 0.0)
            h = jnp.maximum(_mm(h, w3_ref[...]) + b3_ref[...], 0.0)
            g = jnp.maximum(g, jnp.where(d2i <= R2, h, -jnp.inf))
        o_ref[0] = g

    return pl.pallas_call(
        body,
        grid=grid,
        in_specs=[
            pl.BlockSpec((1, Mb, 3), lambda b, i: (b, i, 0)),
            pl.BlockSpec((1, 3, Nn), lambda b, i: (b, 0, 0)),
            pl.BlockSpec((3, C), lambda b, i: (0, 0)),
            pl.BlockSpec((1, C), lambda b, i: (0, 0)),
            pl.BlockSpec((3, C), lambda b, i: (0, 0)),
            pl.BlockSpec((C, C), lambda b, i: (0, 0)),
            pl.BlockSpec((1, C), lambda b, i: (0, 0)),
            pl.BlockSpec((C, C), lambda b, i: (0, 0)),
            pl.BlockSpec((1, C), lambda b, i: (0, 0)),
            pl.BlockSpec((C, C), lambda b, i: (0, 0)),
            pl.BlockSpec((1, C), lambda b, i: (0, 0)),
        ],
        out_specs=pl.BlockSpec((1, Mb, C), lambda b, i: (b, i, 0)),
        out_shape=jax.ShapeDtypeStruct((B, Nn, C), jnp.float32),
        interpret=_INTERP,
    )(x, kxt, w0, b0, w1r, w1f, b1, w2, b2, w3, b3)


def _gc_stage(xyz, feat, layers):
    B, M, C = feat.shape
    w1 = layers[0]['W']
    w1x, w1f = w1[:3], w1[3:]
    H = w1.shape[1]
    b1 = layers[0]['b'].reshape(1, H)
    w2 = layers[1]['W']
    Co = w2.shape[1]
    b2 = layers[1]['b'].reshape(1, Co)

    def body(x_ref, f_ref, w1x_ref, w1f_ref, b1_ref, w2_ref, b2_ref, o_ref):
        e = jax.nn.gelu(_mm(x_ref[0], w1x_ref[...]) + _mm(f_ref[0], w1f_ref[...])
                        + b1_ref[...])
        e = jax.nn.gelu(_mm(e, w2_ref[...]) + b2_ref[...])
        o_ref[0] = jnp.max(e, axis=0, keepdims=True)

    return pl.pallas_call(
        body,
        grid=(B,),
        in_specs=[
            pl.BlockSpec((1, M, 3), lambda b: (b, 0, 0)),
            pl.BlockSpec((1, M, C), lambda b: (b, 0, 0)),
            pl.BlockSpec((3, H), lambda b: (0, 0)),
            pl.BlockSpec((C, H), lambda b: (0, 0)),
            pl.BlockSpec((1, H), lambda b: (0, 0)),
            pl.BlockSpec((H, Co), lambda b: (0, 0)),
            pl.BlockSpec((1, Co), lambda b: (0, 0)),
        ],
        out_specs=pl.BlockSpec((1, 1, Co), lambda b: (b, 0, 0)),
        out_shape=jax.ShapeDtypeStruct((B, 1, Co), jnp.float32),
        interpret=_INTERP,
    )(xyz, feat, w1x, w1f, b1, w2, b2)


def _head_stage(l1_xyz, l1_feat, emb, params):
    B, Nn, Cf = l1_feat.shape
    Ce = emb.shape[-1]
    w = params['up_conv1']['W']
    wx = w[:3]
    wf = w[3:3 + Cf]
    we = w[3 + Cf:]
    H = w.shape[1]
    bias = params['up_conv1']['b'].reshape(1, H)
    w1 = params['se']['W1']
    w2 = params['se']['W2']
    wl = params['last']['W']
    Cl = wl.shape[1]

    def body(x_ref, f_ref, e_ref, wx_ref, wf_ref, we_ref, b_ref, w1_ref,
             w2_ref, wl_ref, o_ref):
        embt = _mm(e_ref[0], we_ref[...])
        ft = jax.nn.gelu(_mm(x_ref[0], wx_ref[...]) + _mm(f_ref[0], wf_ref[...])
                         + embt + b_ref[...])
        s = jnp.mean(ft, axis=0, keepdims=True)
        s = jax.nn.gelu(_mm(s, w1_ref[...]))
        s = jax.nn.sigmoid(_mm(s, w2_ref[...]))
        o_ref[0] = _mm(ft * s, wl_ref[...])

    return pl.pallas_call(
        body,
        grid=(B,),
        in_specs=[
            pl.BlockSpec((1, Nn, 3), lambda b: (b, 0, 0)),
            pl.BlockSpec((1, Nn, Cf), lambda b: (b, 0, 0)),
            pl.BlockSpec((1, 1, Ce), lambda b: (b, 0, 0)),
            pl.BlockSpec((3, H), lambda b: (0, 0)),
            pl.BlockSpec((Cf, H), lambda b: (0, 0)),
            pl.BlockSpec((Ce, H), lambda b: (0, 0)),
            pl.BlockSpec((1, H), lambda b: (0, 0)),
            pl.BlockSpec((H, w1.shape[1]), lambda b: (0, 0)),
            pl.BlockSpec((w1.shape[1], H), lambda b: (0, 0)),
            pl.BlockSpec((H, Cl), lambda b: (0, 0)),
        ],
        out_specs=pl.BlockSpec((1, Nn, Cl), lambda b: (b, 0, 0)),
        out_shape=jax.ShapeDtypeStruct((B, Nn, Cl), jnp.float32),
        interpret=_INTERP,
    )(l1_xyz, l1_feat, emb, wx, wf, we, bias, w1, w2, wl)


def kernel(x, l, params):
    B, N, _ = x.shape
    xyz = x
    r = 0.005
    l1_feat = _sa_stage(x, params)
    l1_xyz = xyz
    l1_feat, l1_xyz = _diffconv_stage(l1_feat, l1_xyz, N, params['conv1'], r)
    l2_feat, l2_xyz = _diffconv_stage(l1_feat, l1_xyz, N // 4, params['conv2'], 4 * r)
    l3_feat, l3_xyz = _diffconv_stage(l2_feat, l2_xyz, N // 8, params['conv3'], 8 * r)
    l4_feat, l4_xyz = _diffconv_stage(l3_feat, l3_xyz, N // 16, params['conv4'], 16 * r)
    l5_feat, l5_xyz = _diffconv_stage(l4_feat, l4_xyz, N // 32, params['conv5'], 32 * r)
    emb1 = _gc_stage(l5_xyz, l5_feat, params['gc1'])
    emb2 = _gc_stage(l4_xyz, l4_feat, params['gc2'])
    emb = jnp.concatenate([emb1, emb2, l.reshape(B, 1, -1)], axis=-1)
    l4_feat = _fp_stage(l4_xyz, l5_xyz, l4_feat, l5_feat, params['fp4'])
    l4_feat, l4_xyz = _diffconv_stage(l4_feat, l4_xyz, N // 16, params['up_conv5'], 16 * r)
    l3_feat = _fp_stage(l3_xyz, l4_xyz, l3_feat, l4_feat, params['fp3'])
    l3_feat, l3_xyz = _diffconv_stage(l3_feat, l3_xyz, N // 8, params['up_conv4'], 8 * r)
    l2_feat = _fp_stage(l2_xyz, l3_xyz, l2_feat, l3_feat, params['fp2'])
    l2_feat, l2_xyz = _diffconv_stage(l2_feat, l2_xyz, N // 4, params['up_conv3'], 4 * r)
    l1_feat = _fp_stage(l1_xyz, l2_xyz, l1_feat, l2_feat, params['fp1'])
    l1_feat, l1_xyz = _diffconv_stage(l1_feat, l1_xyz, N, params['up_conv2'], r)
    out = _head_stage(l1_xyz, l1_feat, emb, params)
    return jnp.transpose(out, (0, 2, 1))


# speedup vs baseline: 25.3034x; 1.9259x over previous
import jax
import jax.numpy as jnp
from jax.experimental import pallas as pl

_INTERP = False
_INF = float('inf')


def _pack(d2):
    bits = jax.lax.bitcast_convert_type(jnp.maximum(d2, 0.0), jnp.int32)
    idx = jax.lax.broadcasted_iota(jnp.int32, d2.shape, d2.ndim - 1)
    return jax.lax.bitcast_convert_type((bits & ~0x7FF) | idx, jnp.float32)


def _extract_min(P):
    m = jnp.min(P, axis=-1, keepdims=True)
    oh = P == m
    return m, oh, jnp.where(oh, _INF, P)


def _select_topk(P, k):
    for _ in range(k):
        _, _, P = _extract_min(P)
    return P == _INF


def _d2(q, kxt):
    qsq = jnp.sum(q * q, axis=-1, keepdims=True)
    ksq = jnp.sum(kxt * kxt, axis=0, keepdims=True)
    qk = jax.lax.dot_general(q, kxt, (((1,), (0,)), ((), ())),
                             preferred_element_type=jnp.float32)
    return (qsq - 2.0 * qk) + ksq


def _mm(a, b):
    return jnp.dot(a, b, preferred_element_type=jnp.float32)


def _knn_mask_stage(xyz, k=20):
    B, M, _ = xyz.shape
    kxt = jnp.swapaxes(xyz, 1, 2)
    Mb = min(M, 256)

    def body(q_ref, kxt_ref, o_ref):
        d2 = _d2(q_ref[0], kxt_ref[0])
        sel = _select_topk(_pack(d2), k)
        o_ref[0] = sel.astype(jnp.int8)

    return pl.pallas_call(
        body,
        grid=(B, M // Mb),
        in_specs=[
            pl.BlockSpec((1, Mb, 3), lambda b, i: (b, i, 0)),
            pl.BlockSpec((1, 3, M), lambda b, i: (b, 0, 0)),
        ],
        out_specs=pl.BlockSpec((1, Mb, M), lambda b, i: (b, i, 0)),
        out_shape=jax.ShapeDtypeStruct((B, M, M), jnp.int8),
        interpret=_INTERP,
    )(xyz, kxt)


def _diffconv_stage(feat, xyz, npoint, p, radius, selmask):
    B, Nn, C = feat.shape
    stride = Nn // npoint
    new_xyz = xyz[:, ::stride]
    cf = feat[:, ::stride]
    kxt = jnp.swapaxes(xyz, 1, 2)
    Co = p['Wd'].shape[1]
    Mb = min(npoint, 256)
    grid = (B, npoint // Mb)
    wa_t = jnp.swapaxes(p['wa'], 0, 1)
    bias = p['b'].reshape(1, Co)

    def body(q_ref, kxt_ref, feat_ref, cf_ref, sel_ref, wat_ref, wd_ref,
             ws_ref, b_ref, o_ref):
        q = q_ref[0]
        kxt_ = kxt_ref[0]
        ft = feat_ref[0]
        cfb = cf_ref[0]
        sel = sel_ref[0] != 0
        d2 = _d2(q, kxt_)
        srow = jax.lax.dot_general(wat_ref[...], ft, (((1,), (1,)), ((), ())),
                                   preferred_element_type=jnp.float32)
        ml = jnp.where(sel, srow - d2 / radius, -_INF)
        rmax = jnp.max(ml, axis=-1, keepdims=True)
        e = jnp.exp(ml - rmax)
        a = e / jnp.sum(e, axis=-1, keepdims=True)
        agg = _mm(a, ft) - cfb
        out = _mm(agg, wd_ref[...]) + _mm(cfb, ws_ref[...]) + b_ref[...]
        o_ref[0] = jax.nn.gelu(out)

    out = pl.pallas_call(
        body,
        grid=grid,
        in_specs=[
            pl.BlockSpec((1, Mb, 3), lambda b, i: (b, i, 0)),
            pl.BlockSpec((1, 3, Nn), lambda b, i: (b, 0, 0)),
            pl.BlockSpec((1, Nn, C), lambda b, i: (b, 0, 0)),
            pl.BlockSpec((1, Mb, C), lambda b, i: (b, i, 0)),
            pl.BlockSpec((1, Mb, Nn), lambda b, i: (b, i, 0)),
            pl.BlockSpec((1, C), lambda b, i: (0, 0)),
            pl.BlockSpec((C, Co), lambda b, i: (0, 0)),
            pl.BlockSpec((C, Co), lambda b, i: (0, 0)),
            pl.BlockSpec((1, Co), lambda b, i: (0, 0)),
        ],
        out_specs=pl.BlockSpec((1, Mb, Co), lambda b, i: (b, i, 0)),
        out_shape=jax.ShapeDtypeStruct((B, npoint, Co), jnp.float32),
        interpret=_INTERP,
    )(new_xyz, kxt, feat, cf, selmask, wa_t, p['Wd'], p['Ws'], bias)
    return out, new_xyz


def _fp_stage(xyz1, xyz2, feat1, feat2, p):
    B, M, C1 = feat1.shape
    _, N2, C2 = feat2.shape
    kxt = jnp.swapaxes(xyz2, 1, 2)
    Co = p['W'].shape[1]
    w_hi = p['W'][:C2]
    w_lo = p['W'][C2:]
    bias = p['b'].reshape(1, Co)
    Mb = min(M, 256)
    grid = (B, M // Mb)

    def body(q_ref, kxt_ref, f2_ref, f1_ref, whi_ref, wlo_ref, b_ref, o_ref):
        d2 = _d2(q_ref[0], kxt_ref[0])
        sel = _select_topk(_pack(d2), 3)
        w = jnp.where(sel, 1.0 / (d2 + 1e-8), 0.0)
        wn = w / jnp.sum(w, axis=-1, keepdims=True)
        interp = _mm(wn, f2_ref[0])
        out = _mm(interp, whi_ref[...]) + _mm(f1_ref[0], wlo_ref[...]) + b_ref[...]
        o_ref[0] = jax.nn.gelu(out)

    return pl.pallas_call(
        body,
        grid=grid,
        in_specs=[
            pl.BlockSpec((1, Mb, 3), lambda b, i: (b, i, 0)),
            pl.BlockSpec((1, 3, N2), lambda b, i: (b, 0, 0)),
            pl.BlockSpec((1, N2, C2), lambda b, i: (b, 0, 0)),
            pl.BlockSpec((1, Mb, C1), lambda b, i: (b, i, 0)),
            pl.BlockSpec((C2, Co), lambda b, i: (0, 0)),
            pl.BlockSpec((C1, Co), lambda b, i: (0, 0)),
            pl.BlockSpec((1, Co), lambda b, i: (0, 0)),
        ],
        out_specs=pl.BlockSpec((1, Mb, Co), lambda b, i: (b, i, 0)),
        out_shape=jax.ShapeDtypeStruct((B, M, Co), jnp.float32),
        interpret=_INTERP,
    )(xyz1, kxt, feat2, feat1, w_hi, w_lo, bias)


def _sa_stage(x, params):
    B, Nn, _ = x.shape
    kxt = jnp.swapaxes(x, 1, 2)
    le0 = params['le0']
    l1 = params['le1']
    w0 = le0['W']
    b0 = le0['b'].reshape(1, -1)
    w1 = l1[0]['W']
    w1r, w1f = w1[:3], w1[3:]
    b1 = l1[0]['b'].reshape(1, -1)
    w2 = l1[1]['W']
    b2 = l1[1]['b'].reshape(1, -1)
    w3 = l1[2]['W']
    b3 = l1[2]['b'].reshape(1, -1)
    C = w0.shape[1]
    Mb = 256
    grid = (B, Nn // Mb)
    R2 = 0.05 * 0.05

    def body(q_ref, kxt_ref, w0_ref, b0_ref, w1r_ref, w1f_ref, b1_ref,
             w2_ref, b2_ref, w3_ref, b3_ref, o_ref, sel_ref):
        q = q_ref[0]
        kxt_ = kxt_ref[0]
        xw1r = jax.lax.dot_general(kxt_, w1r_ref[...], (((0,), (0,)), ((), ())),
                                   preferred_element_type=jnp.float32)
        feat = jax.nn.gelu(
            jax.lax.dot_general(kxt_, w0_ref[...], (((0,), (0,)), ((), ())),
                                preferred_element_type=jnp.float32) + b0_ref[...])
        U = _mm(feat, w1f_ref[...]) + xw1r + b1_ref[...]
        V = _mm(q, w1r_ref[...])
        P = _pack(_d2(q, kxt_))
        g = jnp.full((Mb, C), -_INF, jnp.float32)
        for _ in range(20):
            m, oh, P = _extract_min(P)
            ohf = jnp.where(oh, 1.0, 0.0)
            Ug = _mm(ohf, U)
            h = jnp.maximum(Ug - V, 0.0)
            h = jnp.maximum(_mm(h, w2_ref[...]) + b2_ref[...], 0.0)
            h = jnp.maximum(_mm(h, w3_ref[...]) + b3_ref[...], 0.0)
            g = jnp.maximum(g, jnp.where(m <= R2, h, -_INF))
        o_ref[0] = g
        sel_ref[0] = (P == _INF).astype(jnp.int8)

    return pl.pallas_call(
        body,
        grid=grid,
        in_specs=[
            pl.BlockSpec((1, Mb, 3), lambda b, i: (b, i, 0)),
            pl.BlockSpec((1, 3, Nn), lambda b, i: (b, 0, 0)),
            pl.BlockSpec((3, C), lambda b, i: (0, 0)),
            pl.BlockSpec((1, C), lambda b, i: (0, 0)),
            pl.BlockSpec((3, C), lambda b, i: (0, 0)),
            pl.BlockSpec((C, C), lambda b, i: (0, 0)),
            pl.BlockSpec((1, C), lambda b, i: (0, 0)),
            pl.BlockSpec((C, C), lambda b, i: (0, 0)),
            pl.BlockSpec((1, C), lambda b, i: (0, 0)),
            pl.BlockSpec((C, C), lambda b, i: (0, 0)),
            pl.BlockSpec((1, C), lambda b, i: (0, 0)),
        ],
        out_specs=[
            pl.BlockSpec((1, Mb, C), lambda b, i: (b, i, 0)),
            pl.BlockSpec((1, Mb, Nn), lambda b, i: (b, i, 0)),
        ],
        out_shape=[
            jax.ShapeDtypeStruct((B, Nn, C), jnp.float32),
            jax.ShapeDtypeStruct((B, Nn, Nn), jnp.int8),
        ],
        interpret=_INTERP,
    )(x, kxt, w0, b0, w1r, w1f, b1, w2, b2, w3, b3)


def _gc_stage(xyz, feat, layers):
    B, M, C = feat.shape
    w1 = layers[0]['W']
    w1x, w1f = w1[:3], w1[3:]
    H = w1.shape[1]
    b1 = layers[0]['b'].reshape(1, H)
    w2 = layers[1]['W']
    Co = w2.shape[1]
    b2 = layers[1]['b'].reshape(1, Co)

    def body(x_ref, f_ref, w1x_ref, w1f_ref, b1_ref, w2_ref, b2_ref, o_ref):
        e = jax.nn.gelu(_mm(x_ref[0], w1x_ref[...]) + _mm(f_ref[0], w1f_ref[...])
                        + b1_ref[...])
        e = jax.nn.gelu(_mm(e, w2_ref[...]) + b2_ref[...])
        o_ref[0] = jnp.max(e, axis=0, keepdims=True)

    return pl.pallas_call(
        body,
        grid=(B,),
        in_specs=[
            pl.BlockSpec((1, M, 3), lambda b: (b, 0, 0)),
            pl.BlockSpec((1, M, C), lambda b: (b, 0, 0)),
            pl.BlockSpec((3, H), lambda b: (0, 0)),
            pl.BlockSpec((C, H), lambda b: (0, 0)),
            pl.BlockSpec((1, H), lambda b: (0, 0)),
            pl.BlockSpec((H, Co), lambda b: (0, 0)),
            pl.BlockSpec((1, Co), lambda b: (0, 0)),
        ],
        out_specs=pl.BlockSpec((1, 1, Co), lambda b: (b, 0, 0)),
        out_shape=jax.ShapeDtypeStruct((B, 1, Co), jnp.float32),
        interpret=_INTERP,
    )(xyz, feat, w1x, w1f, b1, w2, b2)


def _head_stage(l1_xyz, l1_feat, emb, params):
    B, Nn, Cf = l1_feat.shape
    Ce = emb.shape[-1]
    w = params['up_conv1']['W']
    wx = w[:3]
    wf = w[3:3 + Cf]
    we = w[3 + Cf:]
    H = w.shape[1]
    bias = params['up_conv1']['b'].reshape(1, H)
    w1 = params['se']['W1']
    w2 = params['se']['W2']
    wl = params['last']['W']
    Cl = wl.shape[1]

    def body(x_ref, f_ref, e_ref, wx_ref, wf_ref, we_ref, b_ref, w1_ref,
             w2_ref, wl_ref, o_ref):
        embt = _mm(e_ref[0], we_ref[...])
        ft = jax.nn.gelu(_mm(x_ref[0], wx_ref[...]) + _mm(f_ref[0], wf_ref[...])
                         + embt + b_ref[...])
        s = jnp.mean(ft, axis=0, keepdims=True)
        s = jax.nn.gelu(_mm(s, w1_ref[...]))
        s = jax.nn.sigmoid(_mm(s, w2_ref[...]))
        o_ref[0] = _mm(ft * s, wl_ref[...])

    return pl.pallas_call(
        body,
        grid=(B,),
        in_specs=[
            pl.BlockSpec((1, Nn, 3), lambda b: (b, 0, 0)),
            pl.BlockSpec((1, Nn, Cf), lambda b: (b, 0, 0)),
            pl.BlockSpec((1, 1, Ce), lambda b: (b, 0, 0)),
            pl.BlockSpec((3, H), lambda b: (0, 0)),
            pl.BlockSpec((Cf, H), lambda b: (0, 0)),
            pl.BlockSpec((Ce, H), lambda b: (0, 0)),
            pl.BlockSpec((1, H), lambda b: (0, 0)),
            pl.BlockSpec((H, w1.shape[1]), lambda b: (0, 0)),
            pl.BlockSpec((w1.shape[1], H), lambda b: (0, 0)),
            pl.BlockSpec((H, Cl), lambda b: (0, 0)),
        ],
        out_specs=pl.BlockSpec((1, Nn, Cl), lambda b: (b, 0, 0)),
        out_shape=jax.ShapeDtypeStruct((B, Nn, Cl), jnp.float32),
        interpret=_INTERP,
    )(l1_xyz, l1_feat, emb, wx, wf, we, bias, w1, w2, wl)


def kernel(x, l, params):
    B, N, _ = x.shape
    xyz = x
    r = 0.005
    l1_feat, sel2048 = _sa_stage(x, params)
    sel512 = _knn_mask_stage(xyz[:, ::4])
    sel256 = _knn_mask_stage(xyz[:, ::8])
    sel128 = _knn_mask_stage(xyz[:, ::16])
    l1_xyz = xyz
    l1_feat, l1_xyz = _diffconv_stage(l1_feat, l1_xyz, N, params['conv1'], r,
                                      sel2048)
    l2_feat, l2_xyz = _diffconv_stage(l1_feat, l1_xyz, N // 4, params['conv2'],
                                      4 * r, sel2048[:, ::4])
    l3_feat, l3_xyz = _diffconv_stage(l2_feat, l2_xyz, N // 8, params['conv3'],
                                      8 * r, sel512[:, ::2])
    l4_feat, l4_xyz = _diffconv_stage(l3_feat, l3_xyz, N // 16, params['conv4'],
                                      16 * r, sel256[:, ::2])
    l5_feat, l5_xyz = _diffconv_stage(l4_feat, l4_xyz, N // 32, params['conv5'],
                                      32 * r, sel128[:, ::2])
    emb1 = _gc_stage(l5_xyz, l5_feat, params['gc1'])
    emb2 = _gc_stage(l4_xyz, l4_feat, params['gc2'])
    emb = jnp.concatenate([emb1, emb2, l.reshape(B, 1, -1)], axis=-1)
    l4_feat = _fp_stage(l4_xyz, l5_xyz, l4_feat, l5_feat, params['fp4'])
    l4_feat, l4_xyz = _diffconv_stage(l4_feat, l4_xyz, N // 16,
                                      params['up_conv5'], 16 * r, sel128)
    l3_feat = _fp_stage(l3_xyz, l4_xyz, l3_feat, l4_feat, params['fp3'])
    l3_feat, l3_xyz = _diffconv_stage(l3_feat, l3_xyz, N // 8,
                                      params['up_conv4'], 8 * r, sel256)
    l2_feat = _fp_stage(l2_xyz, l3_xyz, l2_feat, l3_feat, params['fp2'])
    l2_feat, l2_xyz = _diffconv_stage(l2_feat, l2_xyz, N // 4,
                                      params['up_conv3'], 4 * r, sel512)
    l1_feat = _fp_stage(l1_xyz, l2_xyz, l1_feat, l2_feat, params['fp1'])
    l1_feat, l1_xyz = _diffconv_stage(l1_feat, l1_xyz, N, params['up_conv2'],
                                      r, sel2048)
    out = _head_stage(l1_xyz, l1_feat, emb, params)
    return jnp.transpose(out, (0, 2, 1))


# speedup vs baseline: 37.5069x; 1.4823x over previous
import functools

import jax
import jax.numpy as jnp
from jax import lax
from jax.experimental import pallas as pl
from jax.experimental.pallas import tpu as pltpu
from jax.experimental.pallas import tpu_sc as plsc

_INTERP = False
_INF = float('inf')


def _pack(d2):
    bits = jax.lax.bitcast_convert_type(jnp.maximum(d2, 0.0), jnp.int32)
    idx = jax.lax.broadcasted_iota(jnp.int32, d2.shape, d2.ndim - 1)
    return jax.lax.bitcast_convert_type((bits & ~0x7FF) | idx, jnp.float32)


def _extract_min(P):
    m = jnp.min(P, axis=-1, keepdims=True)
    oh = P == m
    return m, oh, jnp.where(oh, _INF, P)


def _select_topk(P, k):
    for _ in range(k):
        _, _, P = _extract_min(P)
    return P == _INF


def _d2(q, kxt):
    qsq = jnp.sum(q * q, axis=-1, keepdims=True)
    ksq = jnp.sum(kxt * kxt, axis=0, keepdims=True)
    qk = jax.lax.dot_general(q, kxt, (((1,), (0,)), ((), ())),
                             preferred_element_type=jnp.float32)
    return (qsq - 2.0 * qk) + ksq


def _mm(a, b):
    return jnp.dot(a, b, preferred_element_type=jnp.float32)


def _knn_mask_stage(xyz, k=20):
    B, M, _ = xyz.shape
    kxt = jnp.swapaxes(xyz, 1, 2)
    Mb = min(M, 256)

    def body(q_ref, kxt_ref, o_ref):
        d2 = _d2(q_ref[0], kxt_ref[0])
        sel = _select_topk(_pack(d2), k)
        o_ref[0] = sel.astype(jnp.int8)

    return pl.pallas_call(
        body,
        grid=(B, M // Mb),
        in_specs=[
            pl.BlockSpec((1, Mb, 3), lambda b, i: (b, i, 0)),
            pl.BlockSpec((1, 3, M), lambda b, i: (b, 0, 0)),
        ],
        out_specs=pl.BlockSpec((1, Mb, M), lambda b, i: (b, i, 0)),
        out_shape=jax.ShapeDtypeStruct((B, M, M), jnp.int8),
        interpret=_INTERP,
    )(xyz, kxt)


def _diffconv_stage(feat, xyz, npoint, p, radius, selmask):
    B, Nn, C = feat.shape
    stride = Nn // npoint
    new_xyz = xyz[:, ::stride]
    cf = feat[:, ::stride]
    kxt = jnp.swapaxes(xyz, 1, 2)
    Co = p['Wd'].shape[1]
    Mb = min(npoint, 256)
    grid = (B, npoint // Mb)
    wa_t = jnp.swapaxes(p['wa'], 0, 1)
    bias = p['b'].reshape(1, Co)

    def body(q_ref, kxt_ref, feat_ref, cf_ref, sel_ref, wat_ref, wd_ref,
             ws_ref, b_ref, o_ref):
        q = q_ref[0]
        kxt_ = kxt_ref[0]
        ft = feat_ref[0]
        cfb = cf_ref[0]
        sel = sel_ref[0] != 0
        d2 = _d2(q, kxt_)
        srow = jax.lax.dot_general(wat_ref[...], ft, (((1,), (1,)), ((), ())),
                                   preferred_element_type=jnp.float32)
        ml = jnp.where(sel, srow - d2 / radius, -_INF)
        rmax = jnp.max(ml, axis=-1, keepdims=True)
        e = jnp.exp(ml - rmax)
        a = e / jnp.sum(e, axis=-1, keepdims=True)
        agg = _mm(a, ft) - cfb
        out = _mm(agg, wd_ref[...]) + _mm(cfb, ws_ref[...]) + b_ref[...]
        o_ref[0] = jax.nn.gelu(out)

    out = pl.pallas_call(
        body,
        grid=grid,
        in_specs=[
            pl.BlockSpec((1, Mb, 3), lambda b, i: (b, i, 0)),
            pl.BlockSpec((1, 3, Nn), lambda b, i: (b, 0, 0)),
            pl.BlockSpec((1, Nn, C), lambda b, i: (b, 0, 0)),
            pl.BlockSpec((1, Mb, C), lambda b, i: (b, i, 0)),
            pl.BlockSpec((1, Mb, Nn), lambda b, i: (b, i, 0)),
            pl.BlockSpec((1, C), lambda b, i: (0, 0)),
            pl.BlockSpec((C, Co), lambda b, i: (0, 0)),
            pl.BlockSpec((C, Co), lambda b, i: (0, 0)),
            pl.BlockSpec((1, Co), lambda b, i: (0, 0)),
        ],
        out_specs=pl.BlockSpec((1, Mb, Co), lambda b, i: (b, i, 0)),
        out_shape=jax.ShapeDtypeStruct((B, npoint, Co), jnp.float32),
        interpret=_INTERP,
    )(new_xyz, kxt, feat, cf, selmask, wa_t, p['Wd'], p['Ws'], bias)
    return out, new_xyz


def _fp_stage(xyz1, xyz2, feat1, feat2, p):
    B, M, C1 = feat1.shape
    _, N2, C2 = feat2.shape
    kxt = jnp.swapaxes(xyz2, 1, 2)
    Co = p['W'].shape[1]
    w_hi = p['W'][:C2]
    w_lo = p['W'][C2:]
    bias = p['b'].reshape(1, Co)
    Mb = min(M, 256)
    grid = (B, M // Mb)

    def body(q_ref, kxt_ref, f2_ref, f1_ref, whi_ref, wlo_ref, b_ref, o_ref):
        d2 = _d2(q_ref[0], kxt_ref[0])
        sel = _select_topk(_pack(d2), 3)
        w = jnp.where(sel, 1.0 / (d2 + 1e-8), 0.0)
        wn = w / jnp.sum(w, axis=-1, keepdims=True)
        interp = _mm(wn, f2_ref[0])
        out = _mm(interp, whi_ref[...]) + _mm(f1_ref[0], wlo_ref[...]) + b_ref[...]
        o_ref[0] = jax.nn.gelu(out)

    return pl.pallas_call(
        body,
        grid=grid,
        in_specs=[
            pl.BlockSpec((1, Mb, 3), lambda b, i: (b, i, 0)),
            pl.BlockSpec((1, 3, N2), lambda b, i: (b, 0, 0)),
            pl.BlockSpec((1, N2, C2), lambda b, i: (b, 0, 0)),
            pl.BlockSpec((1, Mb, C1), lambda b, i: (b, i, 0)),
            pl.BlockSpec((C2, Co), lambda b, i: (0, 0)),
            pl.BlockSpec((C1, Co), lambda b, i: (0, 0)),
            pl.BlockSpec((1, Co), lambda b, i: (0, 0)),
        ],
        out_specs=pl.BlockSpec((1, Mb, Co), lambda b, i: (b, i, 0)),
        out_shape=jax.ShapeDtypeStruct((B, M, Co), jnp.float32),
        interpret=_INTERP,
    )(xyz1, kxt, feat2, feat1, w_hi, w_lo, bias)


def _sa_utable(x, params):
    B, Nn, _ = x.shape
    kxt = jnp.swapaxes(x, 1, 2)
    le0 = params['le0']
    l1 = params['le1']
    w0 = le0['W']
    b0 = le0['b'].reshape(1, -1)
    w1 = l1[0]['W']
    w1r, w1f = w1[:3], w1[3:]
    b1 = l1[0]['b'].reshape(1, -1)
    C = w0.shape[1]

    def body(kxt_ref, w0_ref, b0_ref, w1r_ref, w1f_ref, b1_ref, o_ref):
        kxt_ = kxt_ref[0]
        xw1r = jax.lax.dot_general(kxt_, w1r_ref[...], (((0,), (0,)), ((), ())),
                                   preferred_element_type=jnp.float32)
        feat = jax.nn.gelu(
            jax.lax.dot_general(kxt_, w0_ref[...], (((0,), (0,)), ((), ())),
                                preferred_element_type=jnp.float32) + b0_ref[...])
        o_ref[0] = _mm(feat, w1f_ref[...]) + xw1r + b1_ref[...]

    return pl.pallas_call(
        body,
        grid=(B,),
        in_specs=[
            pl.BlockSpec((1, 3, Nn), lambda b: (b, 0, 0)),
            pl.BlockSpec((3, C), lambda b: (0, 0)),
            pl.BlockSpec((1, C), lambda b: (0, 0)),
            pl.BlockSpec((3, C), lambda b: (0, 0)),
            pl.BlockSpec((C, C), lambda b: (0, 0)),
            pl.BlockSpec((1, C), lambda b: (0, 0)),
        ],
        out_specs=pl.BlockSpec((1, Nn, C), lambda b: (b, 0, 0)),
        out_shape=jax.ShapeDtypeStruct((B, Nn, C), jnp.float32),
        interpret=_INTERP,
    )(kxt, w0, b0, w1r, w1f, b1)


def _sa_extract(x):
    B, Nn, _ = x.shape
    kxt = jnp.swapaxes(x, 1, 2)
    Mb = 256
    grid = (B, Nn // Mb)

    def body(q_ref, kxt_ref, keys_ref, sel_ref):
        P = _pack(_d2(q_ref[0], kxt_ref[0]))
        for s in range(20):
            m, _, P = _extract_min(P)
            keys_ref[0, :, s:s + 1] = m
        sel_ref[0] = (P == _INF).astype(jnp.int8)

    return pl.pallas_call(
        body,
        grid=grid,
        in_specs=[
            pl.BlockSpec((1, Mb, 3), lambda b, i: (b, i, 0)),
            pl.BlockSpec((1, 3, Nn), lambda b, i: (b, 0, 0)),
        ],
        out_specs=[
            pl.BlockSpec((1, Mb, 20), lambda b, i: (b, i, 0)),
            pl.BlockSpec((1, Mb, Nn), lambda b, i: (b, i, 0)),
        ],
        out_shape=[
            jax.ShapeDtypeStruct((B, Nn, 20), jnp.float32),
            jax.ShapeDtypeStruct((B, Nn, Nn), jnp.int8),
        ],
        interpret=_INTERP,
    )(x, kxt)


def _sc_gather(table, keys_i, n_per_batch):
    T = keys_i.shape[0]
    R, C = table.shape
    NW = 32
    per_w = T // NW
    CH = 2048
    n_ch = per_w // CH
    w_per_batch = n_per_batch // per_w
    rows_per_batch = R // (T // n_per_batch)
    keys3d = keys_i.reshape(T // 1024, 8, 128)
    mesh = plsc.VectorSubcoreMesh(core_axis_name="c", subcore_axis_name="s")

    @functools.partial(
        pl.kernel, mesh=mesh,
        out_type=jax.ShapeDtypeStruct((T, C), jnp.float32),
        compiler_params=pltpu.CompilerParams(use_tc_tiling_on_sc=False),
        scratch_types=[
            pltpu.VMEM((CH // 1024, 8, 128), jnp.int32),
            pltpu.VMEM((CH, C), jnp.float32),
            pltpu.SemaphoreType.DMA,
        ],
    )
    def k(table_hbm, keys_hbm, out_hbm, kv, rows, sem):
        wid = lax.axis_index("s") * 2 + lax.axis_index("c")
        base = wid * per_w
        boff = (wid // w_per_batch) * rows_per_batch
        for c in range(n_ch):
            off = base + c * CH
            pltpu.sync_copy(keys_hbm.at[pl.ds(off // 1024, CH // 1024)], kv)
            for d0 in range(CH // 1024):
                for j in range(8):
                    for lsub in range(8):
                        v = kv[d0, j, pl.ds(lsub * 16, 16)]
                        kv[d0, j, pl.ds(lsub * 16, 16)] = (v & 0x7FF) + boff
            copies = [
                pltpu.make_async_copy(table_hbm.at[kv.at[d // 8, d % 8]],
                                      rows.at[pl.ds(d * 128, 128)], sem)
                for d in range(16)
            ]
            for cp in copies:
                cp.start()
            for cp in copies:
                cp.wait()
            pltpu.sync_copy(rows, out_hbm.at[pl.ds(off, CH)])

    return k(table, keys3d)


def _sa_mlp(x, keys, ug2, params):
    B, Nn, _ = x.shape
    l1 = params['le1']
    w1r = l1[0]['W'][:3]
    w2 = l1[1]['W']
    b2 = l1[1]['b'].reshape(1, -1)
    w3 = l1[2]['W']
    b3 = l1[2]['b'].reshape(1, -1)
    C = w2.shape[0]
    Mb = 256
    grid = (B, Nn // Mb)
    R2 = 0.05 * 0.05

    def body(q_ref, keys_ref, ug_ref, w1r_ref, w2_ref, b2_ref, w3_ref,
             b3_ref, o_ref):
        V = _mm(q_ref[0], w1r_ref[...])
        ug = ug_ref[0]
        keys2 = keys_ref[0]
        g = jnp.full((Mb, C), -_INF, jnp.float32)
        for s in range(20):
            h = jnp.maximum(ug[:, s * C:(s + 1) * C] - V, 0.0)
            h = jnp.maximum(_mm(h, w2_ref[...]) + b2_ref[...], 0.0)
            h = jnp.maximum(_mm(h, w3_ref[...]) + b3_ref[...], 0.0)
            g = jnp.maximum(g, jnp.where(keys2[:, s:s + 1] <= R2, h, -_INF))
        o_ref[0] = g

    return pl.pallas_call(
        body,
        grid=grid,
        in_specs=[
            pl.BlockSpec((1, Mb, 3), lambda b, i: (b, i, 0)),
            pl.BlockSpec((1, Mb, 20), lambda b, i: (b, i, 0)),
            pl.BlockSpec((1, Mb, 20 * C), lambda b, i: (b, i, 0)),
            pl.BlockSpec((3, C), lambda b, i: (0, 0)),
            pl.BlockSpec((C, C), lambda b, i: (0, 0)),
            pl.BlockSpec((1, C), lambda b, i: (0, 0)),
            pl.BlockSpec((C, C), lambda b, i: (0, 0)),
            pl.BlockSpec((1, C), lambda b, i: (0, 0)),
        ],
        out_specs=pl.BlockSpec((1, Mb, C), lambda b, i: (b, i, 0)),
        out_shape=jax.ShapeDtypeStruct((B, Nn, C), jnp.float32),
        interpret=_INTERP,
    )(x, keys, ug2, w1r, w2, b2, w3, b3)


def _sa_stage(x, params):
    B, Nn, _ = x.shape
    C = params['le0']['W'].shape[1]
    utab = _sa_utable(x, params)
    keys, sel = _sa_extract(x)
    keys_i = jax.lax.bitcast_convert_type(keys, jnp.int32).reshape(B * Nn * 20)
    ug = _sc_gather(utab.reshape(B * Nn, C), keys_i, n_per_batch=Nn * 20)
    ug2 = ug.reshape(B, Nn, 20 * C)
    out = _sa_mlp(x, keys, ug2, params)
    return out, sel


def _gc_stage(xyz, feat, layers):
    B, M, C = feat.shape
    w1 = layers[0]['W']
    w1x, w1f = w1[:3], w1[3:]
    H = w1.shape[1]
    b1 = layers[0]['b'].reshape(1, H)
    w2 = layers[1]['W']
    Co = w2.shape[1]
    b2 = layers[1]['b'].reshape(1, Co)

    def body(x_ref, f_ref, w1x_ref, w1f_ref, b1_ref, w2_ref, b2_ref, o_ref):
        e = jax.nn.gelu(_mm(x_ref[0], w1x_ref[...]) + _mm(f_ref[0], w1f_ref[...])
                        + b1_ref[...])
        e = jax.nn.gelu(_mm(e, w2_ref[...]) + b2_ref[...])
        o_ref[0] = jnp.max(e, axis=0, keepdims=True)

    return pl.pallas_call(
        body,
        grid=(B,),
        in_specs=[
            pl.BlockSpec((1, M, 3), lambda b: (b, 0, 0)),
            pl.BlockSpec((1, M, C), lambda b: (b, 0, 0)),
            pl.BlockSpec((3, H), lambda b: (0, 0)),
            pl.BlockSpec((C, H), lambda b: (0, 0)),
            pl.BlockSpec((1, H), lambda b: (0, 0)),
            pl.BlockSpec((H, Co), lambda b: (0, 0)),
            pl.BlockSpec((1, Co), lambda b: (0, 0)),
        ],
        out_specs=pl.BlockSpec((1, 1, Co), lambda b: (b, 0, 0)),
        out_shape=jax.ShapeDtypeStruct((B, 1, Co), jnp.float32),
        interpret=_INTERP,
    )(xyz, feat, w1x, w1f, b1, w2, b2)


def _head_stage(l1_xyz, l1_feat, emb, params):
    B, Nn, Cf = l1_feat.shape
    Ce = emb.shape[-1]
    w = params['up_conv1']['W']
    wx = w[:3]
    wf = w[3:3 + Cf]
    we = w[3 + Cf:]
    H = w.shape[1]
    bias = params['up_conv1']['b'].reshape(1, H)
    w1 = params['se']['W1']
    w2 = params['se']['W2']
    wl = params['last']['W']
    Cl = wl.shape[1]

    def body(x_ref, f_ref, e_ref, wx_ref, wf_ref, we_ref, b_ref, w1_ref,
             w2_ref, wl_ref, o_ref):
        embt = _mm(e_ref[0], we_ref[...])
        ft = jax.nn.gelu(_mm(x_ref[0], wx_ref[...]) + _mm(f_ref[0], wf_ref[...])
                         + embt + b_ref[...])
        s = jnp.mean(ft, axis=0, keepdims=True)
        s = jax.nn.gelu(_mm(s, w1_ref[...]))
        s = jax.nn.sigmoid(_mm(s, w2_ref[...]))
        o_ref[0] = _mm(ft * s, wl_ref[...])

    return pl.pallas_call(
        body,
        grid=(B,),
        in_specs=[
            pl.BlockSpec((1, Nn, 3), lambda b: (b, 0, 0)),
            pl.BlockSpec((1, Nn, Cf), lambda b: (b, 0, 0)),
            pl.BlockSpec((1, 1, Ce), lambda b: (b, 0, 0)),
            pl.BlockSpec((3, H), lambda b: (0, 0)),
            pl.BlockSpec((Cf, H), lambda b: (0, 0)),
            pl.BlockSpec((Ce, H), lambda b: (0, 0)),
            pl.BlockSpec((1, H), lambda b: (0, 0)),
            pl.BlockSpec((H, w1.shape[1]), lambda b: (0, 0)),
            pl.BlockSpec((w1.shape[1], H), lambda b: (0, 0)),
            pl.BlockSpec((H, Cl), lambda b: (0, 0)),
        ],
        out_specs=pl.BlockSpec((1, Nn, Cl), lambda b: (b, 0, 0)),
        out_shape=jax.ShapeDtypeStruct((B, Nn, Cl), jnp.float32),
        interpret=_INTERP,
    )(l1_xyz, l1_feat, emb, wx, wf, we, bias, w1, w2, wl)


def kernel(x, l, params):
    B, N, _ = x.shape
    xyz = x
    r = 0.005
    l1_feat, sel2048 = _sa_stage(x, params)
    sel512 = _knn_mask_stage(xyz[:, ::4])
    sel256 = _knn_mask_stage(xyz[:, ::8])
    sel128 = _knn_mask_stage(xyz[:, ::16])
    l1_xyz = xyz
    l1_feat, l1_xyz = _diffconv_stage(l1_feat, l1_xyz, N, params['conv1'], r,
                                      sel2048)
    l2_feat, l2_xyz = _diffconv_stage(l1_feat, l1_xyz, N // 4, params['conv2'],
                                      4 * r, sel2048[:, ::4])
    l3_feat, l3_xyz = _diffconv_stage(l2_feat, l2_xyz, N // 8, params['conv3'],
                                      8 * r, sel512[:, ::2])
    l4_feat, l4_xyz = _diffconv_stage(l3_feat, l3_xyz, N // 16, params['conv4'],
                                      16 * r, sel256[:, ::2])
    l5_feat, l5_xyz = _diffconv_stage(l4_feat, l4_xyz, N // 32, params['conv5'],
                                      32 * r, sel128[:, ::2])
    emb1 = _gc_stage(l5_xyz, l5_feat, params['gc1'])
    emb2 = _gc_stage(l4_xyz, l4_feat, params['gc2'])
    emb = jnp.concatenate([emb1, emb2, l.reshape(B, 1, -1)], axis=-1)
    l4_feat = _fp_stage(l4_xyz, l5_xyz, l4_feat, l5_feat, params['fp4'])
    l4_feat, l4_xyz = _diffconv_stage(l4_feat, l4_xyz, N // 16,
                                      params['up_conv5'], 16 * r, sel128)
    l3_feat = _fp_stage(l3_xyz, l4_xyz, l3_feat, l4_feat, params['fp3'])
    l3_feat, l3_xyz = _diffconv_stage(l3_feat, l3_xyz, N // 8,
                                      params['up_conv4'], 8 * r, sel256)
    l2_feat = _fp_stage(l2_xyz, l3_xyz, l2_feat, l3_feat, params['fp2'])
    l2_feat, l2_xyz = _diffconv_stage(l2_feat, l2_xyz, N // 4,
                                      params['up_conv3'], 4 * r, sel512)
    l1_feat = _fp_stage(l1_xyz, l2_xyz, l1_feat, l2_feat, params['fp1'])
    l1_feat, l1_xyz = _diffconv_stage(l1_feat, l1_xyz, N, params['up_conv2'],
                                      r, sel2048)
    out = _head_stage(l1_xyz, l1_feat, emb, params)
    return jnp.transpose(out, (0, 2, 1))


# speedup vs baseline: 37.7047x; 1.0053x over previous
import functools

import jax
import jax.numpy as jnp
from jax import lax
from jax.experimental import pallas as pl
from jax.experimental.pallas import tpu as pltpu
from jax.experimental.pallas import tpu_sc as plsc

_INTERP = False
_INF = float('inf')


def _pack(d2):
    bits = jax.lax.bitcast_convert_type(jnp.maximum(d2, 0.0), jnp.int32)
    idx = jax.lax.broadcasted_iota(jnp.int32, d2.shape, d2.ndim - 1)
    return jax.lax.bitcast_convert_type((bits & ~0x7FF) | idx, jnp.float32)


def _extract_min(P):
    m = jnp.min(P, axis=-1, keepdims=True)
    oh = P == m
    return m, oh, jnp.where(oh, _INF, P)


def _select_topk(P, k):
    for _ in range(k):
        _, _, P = _extract_min(P)
    return P == _INF


def _d2(q, kxt):
    qsq = jnp.sum(q * q, axis=-1, keepdims=True)
    ksq = jnp.sum(kxt * kxt, axis=0, keepdims=True)
    qk = jax.lax.dot_general(q, kxt, (((1,), (0,)), ((), ())),
                             preferred_element_type=jnp.float32)
    return (qsq - 2.0 * qk) + ksq


def _mm(a, b):
    return jnp.dot(a, b, preferred_element_type=jnp.float32)


def _knn_mask_stage(xyz, k=20):
    B, M, _ = xyz.shape
    kxt = jnp.swapaxes(xyz, 1, 2)
    Mb = min(M, 256)

    def body(q_ref, kxt_ref, o_ref):
        d2 = _d2(q_ref[0], kxt_ref[0])
        sel = _select_topk(_pack(d2), k)
        o_ref[0] = sel.astype(jnp.int8)

    return pl.pallas_call(
        body,
        grid=(B, M // Mb),
        in_specs=[
            pl.BlockSpec((1, Mb, 3), lambda b, i: (b, i, 0)),
            pl.BlockSpec((1, 3, M), lambda b, i: (b, 0, 0)),
        ],
        out_specs=pl.BlockSpec((1, Mb, M), lambda b, i: (b, i, 0)),
        out_shape=jax.ShapeDtypeStruct((B, M, M), jnp.int8),
        interpret=_INTERP,
    )(xyz, kxt)


def _diffconv_stage(feat, xyz, npoint, p, radius, selmask):
    B, Nn, C = feat.shape
    stride = Nn // npoint
    new_xyz = xyz[:, ::stride]
    cf = feat[:, ::stride]
    kxt = jnp.swapaxes(xyz, 1, 2)
    Co = p['Wd'].shape[1]
    Mb = min(npoint, 256)
    grid = (B, npoint // Mb)
    wa_t = jnp.swapaxes(p['wa'], 0, 1)
    bias = p['b'].reshape(1, Co)

    def body(q_ref, kxt_ref, feat_ref, cf_ref, sel_ref, wat_ref, wd_ref,
             ws_ref, b_ref, o_ref):
        q = q_ref[0]
        kxt_ = kxt_ref[0]
        ft = feat_ref[0]
        cfb = cf_ref[0]
        sel = sel_ref[0] != 0
        d2 = _d2(q, kxt_)
        srow = jax.lax.dot_general(wat_ref[...], ft, (((1,), (1,)), ((), ())),
                                   preferred_element_type=jnp.float32)
        ml = jnp.where(sel, srow - d2 / radius, -_INF)
        rmax = jnp.max(ml, axis=-1, keepdims=True)
        e = jnp.exp(ml - rmax)
        a = e / jnp.sum(e, axis=-1, keepdims=True)
        agg = _mm(a, ft) - cfb
        out = _mm(agg, wd_ref[...]) + _mm(cfb, ws_ref[...]) + b_ref[...]
        o_ref[0] = jax.nn.gelu(out)

    out = pl.pallas_call(
        body,
        grid=grid,
        in_specs=[
            pl.BlockSpec((1, Mb, 3), lambda b, i: (b, i, 0)),
            pl.BlockSpec((1, 3, Nn), lambda b, i: (b, 0, 0)),
            pl.BlockSpec((1, Nn, C), lambda b, i: (b, 0, 0)),
            pl.BlockSpec((1, Mb, C), lambda b, i: (b, i, 0)),
            pl.BlockSpec((1, Mb, Nn), lambda b, i: (b, i, 0)),
            pl.BlockSpec((1, C), lambda b, i: (0, 0)),
            pl.BlockSpec((C, Co), lambda b, i: (0, 0)),
            pl.BlockSpec((C, Co), lambda b, i: (0, 0)),
            pl.BlockSpec((1, Co), lambda b, i: (0, 0)),
        ],
        out_specs=pl.BlockSpec((1, Mb, Co), lambda b, i: (b, i, 0)),
        out_shape=jax.ShapeDtypeStruct((B, npoint, Co), jnp.float32),
        interpret=_INTERP,
    )(new_xyz, kxt, feat, cf, selmask, wa_t, p['Wd'], p['Ws'], bias)
    return out, new_xyz


def _fp_stage(xyz1, xyz2, feat1, feat2, p):
    B, M, C1 = feat1.shape
    _, N2, C2 = feat2.shape
    kxt = jnp.swapaxes(xyz2, 1, 2)
    Co = p['W'].shape[1]
    w_hi = p['W'][:C2]
    w_lo = p['W'][C2:]
    bias = p['b'].reshape(1, Co)
    Mb = min(M, 256)
    grid = (B, M // Mb)

    def body(q_ref, kxt_ref, f2_ref, f1_ref, whi_ref, wlo_ref, b_ref, o_ref):
        d2 = _d2(q_ref[0], kxt_ref[0])
        sel = _select_topk(_pack(d2), 3)
        w = jnp.where(sel, 1.0 / (d2 + 1e-8), 0.0)
        wn = w / jnp.sum(w, axis=-1, keepdims=True)
        interp = _mm(wn, f2_ref[0])
        out = _mm(interp, whi_ref[...]) + _mm(f1_ref[0], wlo_ref[...]) + b_ref[...]
        o_ref[0] = jax.nn.gelu(out)

    return pl.pallas_call(
        body,
        grid=grid,
        in_specs=[
            pl.BlockSpec((1, Mb, 3), lambda b, i: (b, i, 0)),
            pl.BlockSpec((1, 3, N2), lambda b, i: (b, 0, 0)),
            pl.BlockSpec((1, N2, C2), lambda b, i: (b, 0, 0)),
            pl.BlockSpec((1, Mb, C1), lambda b, i: (b, i, 0)),
            pl.BlockSpec((C2, Co), lambda b, i: (0, 0)),
            pl.BlockSpec((C1, Co), lambda b, i: (0, 0)),
            pl.BlockSpec((1, Co), lambda b, i: (0, 0)),
        ],
        out_specs=pl.BlockSpec((1, Mb, Co), lambda b, i: (b, i, 0)),
        out_shape=jax.ShapeDtypeStruct((B, M, Co), jnp.float32),
        interpret=_INTERP,
    )(xyz1, kxt, feat2, feat1, w_hi, w_lo, bias)


def _sa_utable(x, params):
    B, Nn, _ = x.shape
    kxt = jnp.swapaxes(x, 1, 2)
    le0 = params['le0']
    l1 = params['le1']
    w0 = le0['W']
    b0 = le0['b'].reshape(1, -1)
    w1 = l1[0]['W']
    w1r, w1f = w1[:3], w1[3:]
    b1 = l1[0]['b'].reshape(1, -1)
    C = w0.shape[1]

    def body(kxt_ref, w0_ref, b0_ref, w1r_ref, w1f_ref, b1_ref, o_ref):
        kxt_ = kxt_ref[0]
        xw1r = jax.lax.dot_general(kxt_, w1r_ref[...], (((0,), (0,)), ((), ())),
                                   preferred_element_type=jnp.float32)
        feat = jax.nn.gelu(
            jax.lax.dot_general(kxt_, w0_ref[...], (((0,), (0,)), ((), ())),
                                preferred_element_type=jnp.float32) + b0_ref[...])
        o_ref[0] = _mm(feat, w1f_ref[...]) + xw1r + b1_ref[...]

    return pl.pallas_call(
        body,
        grid=(B,),
        in_specs=[
            pl.BlockSpec((1, 3, Nn), lambda b: (b, 0, 0)),
            pl.BlockSpec((3, C), lambda b: (0, 0)),
            pl.BlockSpec((1, C), lambda b: (0, 0)),
            pl.BlockSpec((3, C), lambda b: (0, 0)),
            pl.BlockSpec((C, C), lambda b: (0, 0)),
            pl.BlockSpec((1, C), lambda b: (0, 0)),
        ],
        out_specs=pl.BlockSpec((1, Nn, C), lambda b: (b, 0, 0)),
        out_shape=jax.ShapeDtypeStruct((B, Nn, C), jnp.float32),
        interpret=_INTERP,
    )(kxt, w0, b0, w1r, w1f, b1)


def _sa_extract(x):
    B, Nn, _ = x.shape
    kxt = jnp.swapaxes(x, 1, 2)
    Mb = 512
    grid = (B, Nn // Mb)

    def body(q_ref, kxt_ref, keys_ref, sel_ref):
        P = _pack(_d2(q_ref[0], kxt_ref[0]))
        for s in range(20):
            m, _, P = _extract_min(P)
            keys_ref[0, :, s:s + 1] = m
        sel_ref[0] = (P == _INF).astype(jnp.int8)

    return pl.pallas_call(
        body,
        grid=grid,
        in_specs=[
            pl.BlockSpec((1, Mb, 3), lambda b, i: (b, i, 0)),
            pl.BlockSpec((1, 3, Nn), lambda b, i: (b, 0, 0)),
        ],
        out_specs=[
            pl.BlockSpec((1, Mb, 20), lambda b, i: (b, i, 0)),
            pl.BlockSpec((1, Mb, Nn), lambda b, i: (b, i, 0)),
        ],
        out_shape=[
            jax.ShapeDtypeStruct((B, Nn, 20), jnp.float32),
            jax.ShapeDtypeStruct((B, Nn, Nn), jnp.int8),
        ],
        interpret=_INTERP,
    )(x, kxt)


def _sc_gather(table, keys_i, n_per_batch):
    T = keys_i.shape[0]
    R, C = table.shape
    NW = 32
    per_w = T // NW
    CH = 2048
    n_ch = per_w // CH
    w_per_batch = n_per_batch // per_w
    rows_per_batch = R // (T // n_per_batch)
    keys3d = keys_i.reshape(T // 1024, 8, 128)
    mesh = plsc.VectorSubcoreMesh(core_axis_name="c", subcore_axis_name="s")

    @functools.partial(
        pl.kernel, mesh=mesh,
        out_type=jax.ShapeDtypeStruct((T, C), jnp.float32),
        compiler_params=pltpu.CompilerParams(use_tc_tiling_on_sc=False),
        scratch_types=[
            pltpu.VMEM((CH // 1024, 8, 128), jnp.int32),
            pltpu.VMEM((CH, C), jnp.float32),
            pltpu.SemaphoreType.DMA,
        ],
    )
    def k(table_hbm, keys_hbm, out_hbm, kv, rows, sem):
        wid = lax.axis_index("s") * 2 + lax.axis_index("c")
        base = wid * per_w
        boff = (wid // w_per_batch) * rows_per_batch
        for c in range(n_ch):
            off = base + c * CH
            pltpu.sync_copy(keys_hbm.at[pl.ds(off // 1024, CH // 1024)], kv)
            for d0 in range(CH // 1024):
                for j in range(8):
                    for lsub in range(8):
                        v = kv[d0, j, pl.ds(lsub * 16, 16)]
                        kv[d0, j, pl.ds(lsub * 16, 16)] = (v & 0x7FF) + boff
            copies = [
                pltpu.make_async_copy(table_hbm.at[kv.at[d // 8, d % 8]],
                                      rows.at[pl.ds(d * 128, 128)], sem)
                for d in range(16)
            ]
            for cp in copies:
                cp.start()
            for cp in copies:
                cp.wait()
            pltpu.sync_copy(rows, out_hbm.at[pl.ds(off, CH)])

    return k(table, keys3d)


def _sa_mlp(x, keys, ug2, params):
    B, Nn, _ = x.shape
    l1 = params['le1']
    w1r = l1[0]['W'][:3]
    w2 = l1[1]['W']
    b2 = l1[1]['b'].reshape(1, -1)
    w3 = l1[2]['W']
    b3 = l1[2]['b'].reshape(1, -1)
    C = w2.shape[0]
    Mb = 256
    grid = (B, Nn // Mb)
    R2 = 0.05 * 0.05

    def body(q_ref, keys_ref, ug_ref, w1r_ref, w2_ref, b2_ref, w3_ref,
             b3_ref, o_ref):
        V = _mm(q_ref[0], w1r_ref[...])
        ug = ug_ref[0]
        keys2 = keys_ref[0]
        g = jnp.full((Mb, C), -_INF, jnp.float32)
        for s in range(20):
            h = jnp.maximum(ug[:, s * C:(s + 1) * C] - V, 0.0)
            h = jnp.maximum(_mm(h, w2_ref[...]) + b2_ref[...], 0.0)
            h = jnp.maximum(_mm(h, w3_ref[...]) + b3_ref[...], 0.0)
            g = jnp.maximum(g, jnp.where(keys2[:, s:s + 1] <= R2, h, -_INF))
        o_ref[0] = g

    return pl.pallas_call(
        body,
        grid=grid,
        in_specs=[
            pl.BlockSpec((1, Mb, 3), lambda b, i: (b, i, 0)),
            pl.BlockSpec((1, Mb, 20), lambda b, i: (b, i, 0)),
            pl.BlockSpec((1, Mb, 20 * C), lambda b, i: (b, i, 0)),
            pl.BlockSpec((3, C), lambda b, i: (0, 0)),
            pl.BlockSpec((C, C), lambda b, i: (0, 0)),
            pl.BlockSpec((1, C), lambda b, i: (0, 0)),
            pl.BlockSpec((C, C), lambda b, i: (0, 0)),
            pl.BlockSpec((1, C), lambda b, i: (0, 0)),
        ],
        out_specs=pl.BlockSpec((1, Mb, C), lambda b, i: (b, i, 0)),
        out_shape=jax.ShapeDtypeStruct((B, Nn, C), jnp.float32),
        interpret=_INTERP,
    )(x, keys, ug2, w1r, w2, b2, w3, b3)


def _sa_stage(x, params):
    B, Nn, _ = x.shape
    C = params['le0']['W'].shape[1]
    utab = _sa_utable(x, params)
    keys, sel = _sa_extract(x)
    keys_i = jax.lax.bitcast_convert_type(keys, jnp.int32).reshape(B * Nn * 20)
    ug = _sc_gather(utab.reshape(B * Nn, C), keys_i, n_per_batch=Nn * 20)
    ug2 = ug.reshape(B, Nn, 20 * C)
    out = _sa_mlp(x, keys, ug2, params)
    return out, sel


def _gc_stage(xyz, feat, layers):
    B, M, C = feat.shape
    w1 = layers[0]['W']
    w1x, w1f = w1[:3], w1[3:]
    H = w1.shape[1]
    b1 = layers[0]['b'].reshape(1, H)
    w2 = layers[1]['W']
    Co = w2.shape[1]
    b2 = layers[1]['b'].reshape(1, Co)

    def body(x_ref, f_ref, w1x_ref, w1f_ref, b1_ref, w2_ref, b2_ref, o_ref):
        e = jax.nn.gelu(_mm(x_ref[0], w1x_ref[...]) + _mm(f_ref[0], w1f_ref[...])
                        + b1_ref[...])
        e = jax.nn.gelu(_mm(e, w2_ref[...]) + b2_ref[...])
        o_ref[0] = jnp.max(e, axis=0, keepdims=True)

    return pl.pallas_call(
        body,
        grid=(B,),
        in_specs=[
            pl.BlockSpec((1, M, 3), lambda b: (b, 0, 0)),
            pl.BlockSpec((1, M, C), lambda b: (b, 0, 0)),
            pl.BlockSpec((3, H), lambda b: (0, 0)),
            pl.BlockSpec((C, H), lambda b: (0, 0)),
            pl.BlockSpec((1, H), lambda b: (0, 0)),
            pl.BlockSpec((H, Co), lambda b: (0, 0)),
            pl.BlockSpec((1, Co), lambda b: (0, 0)),
        ],
        out_specs=pl.BlockSpec((1, 1, Co), lambda b: (b, 0, 0)),
        out_shape=jax.ShapeDtypeStruct((B, 1, Co), jnp.float32),
        interpret=_INTERP,
    )(xyz, feat, w1x, w1f, b1, w2, b2)


def _head_stage(l1_xyz, l1_feat, emb, params):
    B, Nn, Cf = l1_feat.shape
    Ce = emb.shape[-1]
    w = params['up_conv1']['W']
    wx = w[:3]
    wf = w[3:3 + Cf]
    we = w[3 + Cf:]
    H = w.shape[1]
    bias = params['up_conv1']['b'].reshape(1, H)
    w1 = params['se']['W1']
    w2 = params['se']['W2']
    wl = params['last']['W']
    Cl = wl.shape[1]

    def body(x_ref, f_ref, e_ref, wx_ref, wf_ref, we_ref, b_ref, w1_ref,
             w2_ref, wl_ref, o_ref):
        embt = _mm(e_ref[0], we_ref[...])
        ft = jax.nn.gelu(_mm(x_ref[0], wx_ref[...]) + _mm(f_ref[0], wf_ref[...])
                         + embt + b_ref[...])
        s = jnp.mean(ft, axis=0, keepdims=True)
        s = jax.nn.gelu(_mm(s, w1_ref[...]))
        s = jax.nn.sigmoid(_mm(s, w2_ref[...]))
        o_ref[0] = _mm(ft * s, wl_ref[...])

    return pl.pallas_call(
        body,
        grid=(B,),
        in_specs=[
            pl.BlockSpec((1, Nn, 3), lambda b: (b, 0, 0)),
            pl.BlockSpec((1, Nn, Cf), lambda b: (b, 0, 0)),
            pl.BlockSpec((1, 1, Ce), lambda b: (b, 0, 0)),
            pl.BlockSpec((3, H), lambda b: (0, 0)),
            pl.BlockSpec((Cf, H), lambda b: (0, 0)),
            pl.BlockSpec((Ce, H), lambda b: (0, 0)),
            pl.BlockSpec((1, H), lambda b: (0, 0)),
            pl.BlockSpec((H, w1.shape[1]), lambda b: (0, 0)),
            pl.BlockSpec((w1.shape[1], H), lambda b: (0, 0)),
            pl.BlockSpec((H, Cl), lambda b: (0, 0)),
        ],
        out_specs=pl.BlockSpec((1, Nn, Cl), lambda b: (b, 0, 0)),
        out_shape=jax.ShapeDtypeStruct((B, Nn, Cl), jnp.float32),
        interpret=_INTERP,
    )(l1_xyz, l1_feat, emb, wx, wf, we, bias, w1, w2, wl)


def kernel(x, l, params):
    B, N, _ = x.shape
    xyz = x
    r = 0.005
    l1_feat, sel2048 = _sa_stage(x, params)
    sel512 = _knn_mask_stage(xyz[:, ::4])
    sel256 = _knn_mask_stage(xyz[:, ::8])
    sel128 = _knn_mask_stage(xyz[:, ::16])
    l1_xyz = xyz
    l1_feat, l1_xyz = _diffconv_stage(l1_feat, l1_xyz, N, params['conv1'], r,
                                      sel2048)
    l2_feat, l2_xyz = _diffconv_stage(l1_feat, l1_xyz, N // 4, params['conv2'],
                                      4 * r, sel2048[:, ::4])
    l3_feat, l3_xyz = _diffconv_stage(l2_feat, l2_xyz, N // 8, params['conv3'],
                                      8 * r, sel512[:, ::2])
    l4_feat, l4_xyz = _diffconv_stage(l3_feat, l3_xyz, N // 16, params['conv4'],
                                      16 * r, sel256[:, ::2])
    l5_feat, l5_xyz = _diffconv_stage(l4_feat, l4_xyz, N // 32, params['conv5'],
                                      32 * r, sel128[:, ::2])
    emb1 = _gc_stage(l5_xyz, l5_feat, params['gc1'])
    emb2 = _gc_stage(l4_xyz, l4_feat, params['gc2'])
    emb = jnp.concatenate([emb1, emb2, l.reshape(B, 1, -1)], axis=-1)
    l4_feat = _fp_stage(l4_xyz, l5_xyz, l4_feat, l5_feat, params['fp4'])
    l4_feat, l4_xyz = _diffconv_stage(l4_feat, l4_xyz, N // 16,
                                      params['up_conv5'], 16 * r, sel128)
    l3_feat = _fp_stage(l3_xyz, l4_xyz, l3_feat, l4_feat, params['fp3'])
    l3_feat, l3_xyz = _diffconv_stage(l3_feat, l3_xyz, N // 8,
                                      params['up_conv4'], 8 * r, sel256)
    l2_feat = _fp_stage(l2_xyz, l3_xyz, l2_feat, l3_feat, params['fp2'])
    l2_feat, l2_xyz = _diffconv_stage(l2_feat, l2_xyz, N // 4,
                                      params['up_conv3'], 4 * r, sel512)
    l1_feat = _fp_stage(l1_xyz, l2_xyz, l1_feat, l2_feat, params['fp1'])
    l1_feat, l1_xyz = _diffconv_stage(l1_feat, l1_xyz, N, params['up_conv2'],
                                      r, sel2048)
    out = _head_stage(l1_xyz, l1_feat, emb, params)
    return jnp.transpose(out, (0, 2, 1))
